# baseline (speedup 1.0000x reference)
import jax
import jax.numpy as jnp
from jax import lax
from jax.experimental import pallas as pl
from jax.experimental.pallas import tpu as pltpu

_N = 10000
_NIN = 128
_NH = 64
_BM = 200
_BMS = 2000


def _agg_body(adj_ref, seq1_ref, seq2_ref, w_ref, b_ref, a_ref, mskc_ref,
              h_ref, red_ref, fts_ref):
    i = pl.program_id(0)

    @pl.when(i == 0)
    def _prologue():
        w = w_ref[...]
        dn = (((1,), (1,)), ((), ()))
        fts_ref[:, :_NH] = lax.dot_general(
            seq1_ref[...], w, dn, preferred_element_type=jnp.float32)
        fts_ref[:, _NH:] = lax.dot_general(
            seq2_ref[...], w, dn, preferred_element_type=jnp.float32)
        red_ref[...] = jnp.zeros((1, 2 * _NH), jnp.float32)

    out = jnp.dot(adj_ref[...], fts_ref[...], preferred_element_type=jnp.float32)
    b = b_ref[...]
    a = a_ref[0, 0]
    o1 = out[:, :_NH] + b
    o2 = out[:, _NH:] + b
    h1 = jnp.where(o1 > 0, o1, a * o1)
    h2 = jnp.where(o2 > 0, o2, a * o2)
    h_ref[:, :_NH] = h1
    h_ref[:, _NH:] = h2
    mskc = mskc_ref[...]
    red_ref[:, :_NH] += jnp.sum(h1 * mskc, axis=0, keepdims=True)
    red_ref[:, _NH:] += jnp.sum(
        jnp.broadcast_to(mskc, (_BM, _NH)), axis=0, keepdims=True)


def _score_body(h_ref, red_ref, wbil_ref, bbil_ref, sb1_ref, sb2_ref, sc_ref):
    red = red_ref[...]
    c = jax.nn.sigmoid(red[0:1, :_NH] / red[0, _NH])
    h = h_ref[...]
    w = wbil_ref[...]
    g1 = jnp.dot(h[:, :_NH], w, preferred_element_type=jnp.float32)
    g2 = jnp.dot(h[:, _NH:], w, preferred_element_type=jnp.float32)
    b = bbil_ref[0, 0]
    sc_ref[:, 0:1] = jnp.sum(g1 * c, axis=1, keepdims=True) + b + sb1_ref[...]
    sc_ref[:, 1:2] = jnp.sum(g2 * c, axis=1, keepdims=True) + b + sb2_ref[...]


def kernel(seq1, seq2, adj, sparse, msk, samp_bias1, samp_bias2,
           W_fc, b_gcn, prelu_a, W_bil, b_bil):
    del sparse
    seq1_2d = seq1.reshape(_N, _NIN)
    seq2_2d = seq2.reshape(_N, _NIN)
    adj_2d = adj.reshape(_N, _N)
    b1r = b_gcn.reshape(1, _NH)
    a11 = prelu_a.reshape(1, 1)
    bbil11 = b_bil.reshape(1, 1)
    mskc = msk.reshape(_N, 1)
    sb1 = samp_bias1.reshape(_N, 1)
    sb2 = samp_bias2.reshape(_N, 1)

    h, red = pl.pallas_call(
        _agg_body,
        grid=(_N // _BM,),
        in_specs=[
            pl.BlockSpec((_BM, _N), lambda i: (i, 0)),
            pl.BlockSpec((_N, _NIN), lambda i: (0, 0)),
            pl.BlockSpec((_N, _NIN), lambda i: (0, 0)),
            pl.BlockSpec((_NH, _NIN), lambda i: (0, 0)),
            pl.BlockSpec((1, _NH), lambda i: (0, 0)),
            pl.BlockSpec((1, 1), lambda i: (0, 0)),
            pl.BlockSpec((_BM, 1), lambda i: (i, 0)),
        ],
        out_specs=[
            pl.BlockSpec((_BM, 2 * _NH), lambda i: (i, 0)),
            pl.BlockSpec((1, 2 * _NH), lambda i: (0, 0)),
        ],
        out_shape=[
            jax.ShapeDtypeStruct((_N, 2 * _NH), jnp.float32),
            jax.ShapeDtypeStruct((1, 2 * _NH), jnp.float32),
        ],
        scratch_shapes=[pltpu.VMEM((_N, 2 * _NH), jnp.float32)],
    )(adj_2d, seq1_2d, seq2_2d, W_fc, b1r, a11, mskc)

    sc = pl.pallas_call(
        _score_body,
        grid=(_N // _BMS,),
        in_specs=[
            pl.BlockSpec((_BMS, 2 * _NH), lambda i: (i, 0)),
            pl.BlockSpec((1, 2 * _NH), lambda i: (0, 0)),
            pl.BlockSpec((_NH, _NH), lambda i: (0, 0)),
            pl.BlockSpec((1, 1), lambda i: (0, 0)),
            pl.BlockSpec((_BMS, 1), lambda i: (i, 0)),
            pl.BlockSpec((_BMS, 1), lambda i: (i, 0)),
        ],
        out_specs=pl.BlockSpec((_BMS, 2), lambda i: (i, 0)),
        out_shape=jax.ShapeDtypeStruct((_N, 2), jnp.float32),
    )(h, red, W_bil.reshape(_NH, _NH), bbil11, sb1, sb2)

    return sc.T.reshape(1, 2 * _N)


# speedup vs baseline: 1.1296x; 1.1296x over previous
import jax
import jax.numpy as jnp
from jax import lax
from jax.experimental import pallas as pl
from jax.experimental.pallas import tpu as pltpu

_N = 10000
_NIN = 128
_NH = 64
_BM = 200
_NB = _N // _BM


def _body(adj_ref, seq1_ref, seq2_ref, w_ref, b_ref, a_ref, mskr_ref,
          wbil_ref, bbil_ref, sc_ref, fts_ref, h_scr):
    i = pl.program_id(0)

    @pl.when(i == 0)
    def _prologue():
        w = w_ref[...]
        dn = (((1,), (1,)), ((), ()))
        fts_ref[:, :_NH] = lax.dot_general(
            seq1_ref[...], w, dn, preferred_element_type=jnp.float32)
        fts_ref[:, _NH:] = lax.dot_general(
            seq2_ref[...], w, dn, preferred_element_type=jnp.float32)

    out = jnp.dot(adj_ref[...], fts_ref[...], preferred_element_type=jnp.float32)
    b = b_ref[...]
    a = a_ref[0, 0]
    o1 = out[:, :_NH] + b
    o2 = out[:, _NH:] + b
    h_scr[pl.ds(i * _BM, _BM), :_NH] = jnp.where(o1 > 0, o1, a * o1)
    h_scr[pl.ds(i * _BM, _BM), _NH:] = jnp.where(o2 > 0, o2, a * o2)

    @pl.when(i == _NB - 1)
    def _epilogue():
        hf1 = h_scr[:, :_NH]
        hf2 = h_scr[:, _NH:]
        mskr = mskr_ref[...]
        mskb = jnp.broadcast_to(mskr, (8, _N))
        csum = lax.dot_general(mskb, hf1, (((1,), (0,)), ((), ())),
                               preferred_element_type=jnp.float32)
        c = jax.nn.sigmoid(csum[0:1, :] / jnp.sum(mskr))
        wb = wbil_ref[...]
        g1 = jnp.dot(hf1, wb, preferred_element_type=jnp.float32)
        g2 = jnp.dot(hf2, wb, preferred_element_type=jnp.float32)
        bb = bbil_ref[0, 0]
        sc_ref[:, 0:1] = jnp.sum(g1 * c, axis=1, keepdims=True) + bb
        sc_ref[:, 1:2] = jnp.sum(g2 * c, axis=1, keepdims=True) + bb


def kernel(seq1, seq2, adj, sparse, msk, samp_bias1, samp_bias2,
           W_fc, b_gcn, prelu_a, W_bil, b_bil):
    del sparse
    seq1_2d = seq1.reshape(_N, _NIN)
    seq2_2d = seq2.reshape(_N, _NIN)
    adj_2d = adj.reshape(_N, _N)
    b1r = b_gcn.reshape(1, _NH)
    a11 = prelu_a.reshape(1, 1)
    bbil11 = b_bil.reshape(1, 1)

    pin = lambda i: (0, 0)
    sc = pl.pallas_call(
        _body,
        grid=(_NB,),
        in_specs=[
            pl.BlockSpec((_BM, _N), lambda i: (i, 0)),
            pl.BlockSpec((_N, _NIN), pin),
            pl.BlockSpec((_N, _NIN), pin),
            pl.BlockSpec((_NH, _NIN), pin),
            pl.BlockSpec((1, _NH), pin),
            pl.BlockSpec((1, 1), pin),
            pl.BlockSpec((1, _N), pin),
            pl.BlockSpec((_NH, _NH), pin),
            pl.BlockSpec((1, 1), pin),
        ],
        out_specs=pl.BlockSpec((_N, 2), pin),
        out_shape=jax.ShapeDtypeStruct((_N, 2), jnp.float32),
        scratch_shapes=[
            pltpu.VMEM((_N, 2 * _NH), jnp.float32),
            pltpu.VMEM((_N, 2 * _NH), jnp.float32),
        ],
    )(adj_2d, seq1_2d, seq2_2d, W_fc, b1r, a11, msk,
      W_bil.reshape(_NH, _NH), bbil11)

    logits = sc.T.reshape(1, 2 * _N)
    return logits + jnp.concatenate([samp_bias1, samp_bias2], axis=1)
